# Initial kernel scaffold; baseline (speedup 1.0000x reference)
import jax
import jax.numpy as jnp
from jax.experimental import pallas as pl


def kernel(layer_in, edge_index, adj_values, W, bias):
    raise NotImplementedError("write your pallas kernel here")


# speedup vs baseline: 6.7804x; 6.7804x over previous
import functools

import jax
import jax.numpy as jnp
from jax import lax
from jax.experimental import pallas as pl
from jax.experimental.pallas import tpu as pltpu
from jax.experimental.pallas import tpu_sc as plsc

NW = 32
NTILE = 16
CHUNK = 128

_mesh = lambda: plsc.VectorSubcoreMesh(core_axis_name="c", subcore_axis_name="s")
_SC_PARAMS = pltpu.CompilerParams(needs_layout_passes=False)


def _rowsum_call(row_p, adj_p, zeros1, n_pad, epw):
    nchunk = epw // CHUNK
    rows_per_tile = n_pad // NTILE

    @functools.partial(
        pl.kernel,
        mesh=_mesh(),
        out_type=jax.ShapeDtypeStruct((2, n_pad), jnp.float32),
        compiler_params=_SC_PARAMS,
        scratch_types=[
            pltpu.VMEM((1, CHUNK), jnp.int32),
            pltpu.VMEM((CHUNK,), jnp.float32),
            pltpu.VMEM_SHARED((n_pad,), jnp.float32),
        ],
    )
    def rowsum_k(row_hbm, adj_hbm, zeros_hbm, rs_out, idx_v, adj_v, rs_sh):
        c = lax.axis_index("c")
        s = lax.axis_index("s")
        wid = s * 2 + c
        base0 = wid * epw
        tslice = pl.ds(s * rows_per_tile, rows_per_tile)
        pltpu.sync_copy(zeros_hbm.at[tslice], rs_sh.at[tslice])
        plsc.subcore_barrier()

        def body(i, carry):
            base = base0 + i * CHUNK
            pltpu.sync_copy(row_hbm.at[pl.ds(base, CHUNK)], idx_v.at[0])
            pltpu.sync_copy(adj_hbm.at[pl.ds(base, CHUNK)], adj_v)
            pltpu.sync_copy(adj_v, rs_sh.at[idx_v.at[0]], add=True)
            return carry

        lax.fori_loop(0, nchunk, body, 0)
        plsc.subcore_barrier()
        pltpu.sync_copy(rs_sh.at[tslice], rs_out.at[c, tslice])

    return rowsum_k(row_p, adj_p, zeros1)


def _matmul_scale_call(x_p, W, rs_parts, n_pad, d_in, d_out):
    bm = 256
    grid = n_pad // bm

    def body(x_ref, w_ref, rs_ref, y_ref):
        rs = rs_ref[0, :] + rs_ref[1, :]
        xw = jnp.dot(x_ref[...], w_ref[...], preferred_element_type=jnp.float32)
        y_ref[...] = xw * (1.0 / rs)[:, None]

    return pl.pallas_call(
        body,
        grid=(grid,),
        in_specs=[
            pl.BlockSpec((bm, d_in), lambda i: (i, 0)),
            pl.BlockSpec((d_in, d_out), lambda i: (0, 0)),
            pl.BlockSpec((2, bm), lambda i: (0, i)),
        ],
        out_specs=pl.BlockSpec((bm, d_out), lambda i: (i, 0)),
        out_shape=jax.ShapeDtypeStruct((n_pad, d_out), jnp.float32),
    )(x_p, W, rs_parts)


def _spmm_call(y, col_p, row_p, adj_p, zeros2, n_pad, d_out, epw):
    nchunk = epw // CHUNK
    rows_per_tile = n_pad // NTILE
    nvec = d_out // 16

    @functools.partial(
        pl.kernel,
        mesh=_mesh(),
        out_type=jax.ShapeDtypeStruct((2, n_pad, d_out), jnp.float32),
        compiler_params=_SC_PARAMS,
        scratch_types=[
            pltpu.VMEM((CHUNK,), jnp.int32),
            pltpu.VMEM((1, CHUNK), jnp.int32),
            pltpu.VMEM((CHUNK,), jnp.float32),
            pltpu.VMEM((CHUNK, d_out), jnp.float32),
            pltpu.VMEM_SHARED((n_pad, d_out), jnp.float32),
            pltpu.SemaphoreType.DMA,
        ],
    )
    def spmm_k(y_hbm, col_hbm, row_hbm, adj_hbm, zeros_hbm, out_hbm,
               colv, rowv, adjv, rows, acc, sem):
        c = lax.axis_index("c")
        s = lax.axis_index("s")
        wid = s * 2 + c
        base0 = wid * epw
        tslice = pl.ds(s * rows_per_tile, rows_per_tile)
        pltpu.sync_copy(zeros_hbm.at[tslice], acc.at[tslice])
        plsc.subcore_barrier()

        def chunk_body(i, carry):
            base = base0 + i * CHUNK
            pltpu.sync_copy(col_hbm.at[pl.ds(base, CHUNK)], colv)
            pltpu.sync_copy(row_hbm.at[pl.ds(base, CHUNK)], rowv.at[0])
            pltpu.sync_copy(adj_hbm.at[pl.ds(base, CHUNK)], adjv)
            pltpu.async_copy(y_hbm.at[colv], rows, sem).wait()

            def scale_body(e, carry2):
                coef = plsc.load_gather(adjv, [jnp.full((16,), e, jnp.int32)])
                for d in range(nvec):
                    sl = pl.ds(d * 16, 16)
                    rows[e, sl] = rows[e, sl] * coef
                return carry2

            lax.fori_loop(0, CHUNK, scale_body, 0)
            pltpu.sync_copy(rows, acc.at[rowv.at[0]], add=True)
            return carry

        lax.fori_loop(0, nchunk, chunk_body, 0)
        plsc.subcore_barrier()
        pltpu.sync_copy(acc.at[tslice], out_hbm.at[c, tslice])

    return spmm_k(y, col_p, row_p, adj_p, zeros2)


def _combine_call(parts, bias2d, n_nodes, d_out):
    bm = 400
    grid = n_nodes // bm

    def body(p_ref, b_ref, o_ref):
        o_ref[...] = p_ref[0] + p_ref[1] + b_ref[...]

    return pl.pallas_call(
        body,
        grid=(grid,),
        in_specs=[
            pl.BlockSpec((2, bm, d_out), lambda i: (0, i, 0)),
            pl.BlockSpec((1, d_out), lambda i: (0, 0)),
        ],
        out_specs=pl.BlockSpec((bm, d_out), lambda i: (i, 0)),
        out_shape=jax.ShapeDtypeStruct((n_nodes, d_out), jnp.float32),
    )(parts, bias2d)


def kernel(layer_in, edge_index, adj_values, W, bias):
    n_nodes, d_in = layer_in.shape
    d_out = W.shape[1]
    n_edges = adj_values.shape[0]

    epw = -(-n_edges // (NW * CHUNK)) * CHUNK
    e_pad = epw * NW - n_edges
    n_pad = -(-n_nodes // 1024) * 1024

    row = edge_index[0]
    col = edge_index[1]
    row_p = jnp.concatenate([row, jnp.full((e_pad,), n_nodes, jnp.int32)])
    col_p = jnp.concatenate([col, jnp.zeros((e_pad,), jnp.int32)])
    adj_p = jnp.concatenate([adj_values, jnp.zeros((e_pad,), jnp.float32)])
    x_p = jnp.pad(layer_in, ((0, n_pad - n_nodes), (0, 0)))
    zeros1 = jnp.zeros((n_pad,), jnp.float32)
    zeros2 = jnp.zeros((n_pad, d_out), jnp.float32)

    rs_parts = _rowsum_call(row_p, adj_p, zeros1, n_pad, epw)
    y = _matmul_scale_call(x_p, W, rs_parts, n_pad, d_in, d_out)
    parts = _spmm_call(y, col_p, row_p, adj_p, zeros2, n_pad, d_out, epw)
    out = _combine_call(parts, bias.reshape(1, d_out), n_nodes, d_out)
    return out
